# Initial kernel scaffold; baseline (speedup 1.0000x reference)
import functools

import jax
import jax.numpy as jnp
from jax import lax
from jax.experimental import pallas as pl
from jax.experimental.pallas import tpu as pltpu
from jax.experimental.pallas import tpu_sc as plsc

NE = 50000
NU = 20000
C = 128
QW = 32
LO, HI = 42033, 44630
RD = HI - LO
RPAD = 2688
NEDGE = 600000
NNZ = 500000
EP = 614400
IP = 512000
ER = EP // 128
IR = IP // 128
ET = ER // 16
IT = IR // 16
ACC_E = 50016
DUM_E = NE
DUM_U = NU
ZR = 3126
UZR = 1251
EF = NE // 16
UF = NU // 16


def _region_blend(rwm_pad, ent_pad):
    def body(a_ref, b_ref, s_ref, o_ref):
        mm = jnp.dot(a_ref[...], b_ref[...],
                     preferred_element_type=jnp.float32)
        o_ref[...] = 0.8 * s_ref[...] + 0.2 * mm

    return pl.pallas_call(
        body,
        grid=(RPAD // 128,),
        in_specs=[
            pl.BlockSpec((128, RPAD), lambda i: (i, 0)),
            pl.BlockSpec((RPAD, 128), lambda i: (0, 0)),
            pl.BlockSpec((128, 128), lambda i: (i, 0)),
        ],
        out_specs=pl.BlockSpec((128, 128), lambda i: (i, 0)),
        out_shape=jax.ShapeDtypeStruct((RPAD, 128), jnp.float32),
    )(rwm_pad, ent_pad, ent_pad)


def _norm_ent(aggq, counts, res_in):
    R = 500

    def body(a_ref, c_ref, r_ref, ro_ref, eo_ref):
        a = a_ref[...]
        x = jnp.concatenate([a[0], a[1], a[2], a[3]], axis=-1)
        cnt = jnp.maximum(c_ref[...][:, 0:1], 1.0)
        x = x / cnt
        n = jnp.sqrt(jnp.sum(x * x, axis=-1, keepdims=True))
        y = x / jnp.maximum(n, 1e-12)
        ro_ref[...] = r_ref[...] + y
        eo_ref[...] = y

    return pl.pallas_call(
        body,
        grid=(NE // R,),
        in_specs=[
            pl.BlockSpec((4, R, 32), lambda i: (0, i, 0)),
            pl.BlockSpec((R, 8), lambda i: (i, 0)),
            pl.BlockSpec((R, 128), lambda i: (i, 0)),
        ],
        out_specs=[pl.BlockSpec((R, 128), lambda i: (i, 0))] * 2,
        out_shape=[jax.ShapeDtypeStruct((NE, 128), jnp.float32)] * 2,
    )(aggq, counts, res_in)


def _norm_user(aggq, res_in):
    R = 500

    def body(a_ref, r_ref, ro_ref):
        a = a_ref[...]
        x = jnp.concatenate([a[0], a[1], a[2], a[3]], axis=-1)
        n = jnp.sqrt(jnp.sum(x * x, axis=-1, keepdims=True))
        y = x / jnp.maximum(n, 1e-12)
        ro_ref[...] = r_ref[...] + y

    return pl.pallas_call(
        body,
        grid=(NU // R,),
        in_specs=[
            pl.BlockSpec((4, R, 32), lambda i: (0, i, 0)),
            pl.BlockSpec((R, 128), lambda i: (i, 0)),
        ],
        out_specs=pl.BlockSpec((R, 128), lambda i: (i, 0)),
        out_shape=jax.ShapeDtypeStruct((NU, 128), jnp.float32),
    )(aggq, res_in)


def _kg_pass(q, count_this, sid, eq, heads, tails, types, wflat, zeros32,
             zeros8, oe, ocnt, acc, accc, hbuf, tbuf, ybuf, rows, wv, onev,
             gsem):
    pltpu.sync_copy(wflat.at[q], wv)
    pltpu.sync_copy(zeros32, acc.at[pl.ds(sid * ZR, ZR)])
    if count_this:
        pltpu.sync_copy(zeros8, accc.at[pl.ds(sid * ZR, ZR)])
    plsc.subcore_barrier()

    def chunk(j, carry):
        row = sid * ET + j
        pltpu.sync_copy(tails.at[pl.ds(row, 1)], tbuf)
        pltpu.sync_copy(heads.at[pl.ds(row, 1)], hbuf)
        pltpu.sync_copy(types.at[pl.ds(row, 1)], ybuf)
        pltpu.async_copy(eq.at[tbuf.at[0]], rows, gsem).wait()

        @plsc.parallel_loop(0, 128, step=1, unroll=8)
        def _mul(i):
            off = (ybuf[0, i] - 1) * QW
            w0 = wv[pl.ds(off, 16)]
            w1 = wv[pl.ds(off + 16, 16)]
            rows[i, pl.ds(0, 16)] = rows[i, pl.ds(0, 16)] * w0
            rows[i, pl.ds(16, 16)] = rows[i, pl.ds(16, 16)] * w1

        pltpu.sync_copy(rows, acc.at[hbuf.at[0]], add=True)
        if count_this:
            pltpu.sync_copy(onev, accc.at[hbuf.at[0]], add=True)
        return carry

    lax.fori_loop(0, ET, chunk, 0)
    plsc.subcore_barrier()
    pltpu.sync_copy(acc.at[pl.ds(sid * EF, EF)], oe.at[pl.ds(sid * EF, EF)])
    if count_this:
        pltpu.sync_copy(accc.at[pl.ds(sid * EF, EF)],
                        ocnt.at[pl.ds(sid * EF, EF)])
    plsc.subcore_barrier()


def _user_pass(q, sid, eq, irows, icols, ivals, zeros32, ou, acc, hbuf, tbuf,
               vbuf, rows, gsem):
    pltpu.sync_copy(zeros32.at[pl.ds(0, UZR)], acc.at[pl.ds(sid * UZR, UZR)])
    plsc.subcore_barrier()

    def chunk(j, carry):
        row = sid * IT + j
        pltpu.sync_copy(icols.at[pl.ds(row, 1)], tbuf)
        pltpu.sync_copy(irows.at[pl.ds(row, 1)], hbuf)
        pltpu.sync_copy(ivals.at[pl.ds(row, 1)], vbuf)
        pltpu.async_copy(eq.at[tbuf.at[0]], rows, gsem).wait()

        @plsc.parallel_loop(0, 128, step=1, unroll=8)
        def _mul(i):
            v = vbuf[0, i]
            rows[i, pl.ds(0, 16)] = rows[i, pl.ds(0, 16)] * v
            rows[i, pl.ds(16, 16)] = rows[i, pl.ds(16, 16)] * v

        pltpu.sync_copy(rows, acc.at[hbuf.at[0]], add=True)
        return carry

    lax.fori_loop(0, IT, chunk, 0)
    plsc.subcore_barrier()
    pltpu.sync_copy(acc.at[pl.ds(sid * UF, UF)], ou.at[pl.ds(sid * UF, UF)])
    plsc.subcore_barrier()


def _sc_body(do_counts, eq0, eq1, eq2, eq3, heads, tails, types, irows,
             icols, ivals, wflat, ones8, zeros32, zeros8,
             oe0, oe1, oe2, oe3, ou0, ou1, ou2, ou3, ocnt,
             acc, accc, hbuf, tbuf, ybuf, vbuf, rows, wv, onev, gsem):
    cid = lax.axis_index("c")
    sid = lax.axis_index("s")
    eqs = (eq0, eq1, eq2, eq3)
    oes = (oe0, oe1, oe2, oe3)
    ous = (ou0, ou1, ou2, ou3)
    pltpu.sync_copy(ones8, onev)

    for q in range(4):
        count_this = do_counts and q == 0

        def _make(qq, cc):
            def _thunk():
                _kg_pass(qq, cc, sid, eqs[qq], heads, tails, types, wflat,
                         zeros32, zeros8, oes[qq], ocnt, acc, accc, hbuf,
                         tbuf, ybuf, rows, wv, onev, gsem)
                _user_pass(qq, sid, eqs[qq], irows, icols, ivals, zeros32,
                           ous[qq], acc, hbuf, tbuf, vbuf, rows, gsem)
            return _thunk

        pl.when(cid == q // 2)(_make(q, count_this))


@functools.lru_cache(maxsize=2)
def _sc_agg(do_counts):
    mesh = plsc.VectorSubcoreMesh(core_axis_name="c", subcore_axis_name="s")
    out_type = (
        [jax.ShapeDtypeStruct((NE, 32), jnp.float32)] * 4
        + [jax.ShapeDtypeStruct((NU, 32), jnp.float32)] * 4
        + [jax.ShapeDtypeStruct((NE, 8), jnp.float32)]
    )
    scratch = [
        pltpu.VMEM_SHARED((ACC_E, 32), jnp.float32),
        pltpu.VMEM_SHARED((ACC_E, 8), jnp.float32),
        pltpu.VMEM((1, 128), jnp.int32),
        pltpu.VMEM((1, 128), jnp.int32),
        pltpu.VMEM((1, 128), jnp.int32),
        pltpu.VMEM((1, 128), jnp.float32),
        pltpu.VMEM((128, 32), jnp.float32),
        pltpu.VMEM((352,), jnp.float32),
        pltpu.VMEM((128, 8), jnp.float32),
        pltpu.SemaphoreType.DMA,
    ]
    return pl.kernel(
        functools.partial(_sc_body, do_counts),
        out_type=out_type,
        mesh=mesh,
        scratch_types=scratch,
    )


def kernel(user_emb, entity_emb, edge_index, edge_type, interact_rows,
           interact_cols, interact_values, region_weight_mat, weight):
    head = edge_index[0]
    tail = edge_index[1]
    pe = EP - NEDGE
    heads_p = jnp.concatenate(
        [head, jnp.full((pe,), DUM_E, jnp.int32)]).reshape(ER, 128)
    tails_p = jnp.concatenate(
        [tail, jnp.zeros((pe,), jnp.int32)]).reshape(ER, 128)
    types_p = jnp.concatenate(
        [edge_type, jnp.ones((pe,), jnp.int32)]).reshape(ER, 128)
    pi = IP - NNZ
    irows_p = jnp.concatenate(
        [interact_rows, jnp.full((pi,), DUM_U, jnp.int32)]).reshape(IR, 128)
    icols_p = jnp.concatenate(
        [interact_cols, jnp.zeros((pi,), jnp.int32)]).reshape(IR, 128)
    ivals_p = jnp.concatenate(
        [interact_values, jnp.zeros((pi,), jnp.float32)]).reshape(IR, 128)
    wflat = weight.reshape(11, 4, 32).transpose(1, 0, 2).reshape(4, 352)
    ones8 = jnp.ones((128, 8), jnp.float32)
    zeros32 = jnp.zeros((ZR, 32), jnp.float32)
    zeros8 = jnp.zeros((ZR, 8), jnp.float32)
    rwm_pad = jnp.pad(region_weight_mat, ((0, RPAD - RD), (0, RPAD - RD)))

    ent = entity_emb
    e_res = entity_emb
    u_res = user_emb
    counts = None
    for hop in range(2):
        ent_pad = jnp.pad(ent[LO:HI], ((0, RPAD - RD), (0, 0)))
        e_region = _region_blend(rwm_pad, ent_pad)[:RD]
        e = jnp.concatenate([ent[:LO], e_region, ent[HI:]], axis=0)
        eq = e.reshape(NE, 4, 32).transpose(1, 0, 2)
        outs = _sc_agg(hop == 0)(
            eq[0], eq[1], eq[2], eq[3], heads_p, tails_p, types_p,
            irows_p, icols_p, ivals_p, wflat, ones8, zeros32, zeros8)
        oe0, oe1, oe2, oe3, ou0, ou1, ou2, ou3, ocnt = outs
        if hop == 0:
            counts = ocnt
        eagg = jnp.stack([oe0, oe1, oe2, oe3])
        uagg = jnp.stack([ou0, ou1, ou2, ou3])
        e_res, ent = _norm_ent(eagg, counts, e_res)
        u_res = _norm_user(uagg, u_res)
    return e_res, u_res


# speedup vs baseline: 1.4761x; 1.4761x over previous
import functools

import jax
import jax.numpy as jnp
from jax import lax
from jax.experimental import pallas as pl
from jax.experimental.pallas import tpu as pltpu
from jax.experimental.pallas import tpu_sc as plsc

NE = 50000
NU = 20000
C = 128
QW = 32
LO, HI = 42033, 44630
RD = HI - LO
RPAD = 2688
NEDGE = 600000
NNZ = 500000
EP = 614400
IP = 512000
ER = EP // 128
IR = IP // 128
ET = ER // 16
IT = IR // 16
ES = 3128
US = 1256
ACC_E = 50008
ACC_U = 20008
DUM_E = NE
DUM_U = NU


def _region_blend(rwm_pad, ent_pad):
    def body(a_ref, b_ref, s_ref, o_ref):
        mm = jnp.dot(a_ref[...], b_ref[...],
                     preferred_element_type=jnp.float32)
        o_ref[...] = 0.8 * s_ref[...] + 0.2 * mm

    return pl.pallas_call(
        body,
        grid=(RPAD // 128,),
        in_specs=[
            pl.BlockSpec((128, RPAD), lambda i: (i, 0)),
            pl.BlockSpec((RPAD, 128), lambda i: (0, 0)),
            pl.BlockSpec((128, 128), lambda i: (i, 0)),
        ],
        out_specs=pl.BlockSpec((128, 128), lambda i: (i, 0)),
        out_shape=jax.ShapeDtypeStruct((RPAD, 128), jnp.float32),
    )(rwm_pad, ent_pad, ent_pad)


def _norm_ent(aggq, counts, res_in):
    R = 400

    def body(a_ref, c_ref, r_ref, ro_ref, eo_ref):
        a = a_ref[...]
        x = jnp.concatenate([a[0], a[1], a[2], a[3]], axis=-1)
        cnt = jnp.maximum(c_ref[...][:, 0:1], 1.0)
        x = x / cnt
        n = jnp.sqrt(jnp.sum(x * x, axis=-1, keepdims=True))
        y = x / jnp.maximum(n, 1e-12)
        ro_ref[...] = r_ref[...] + y
        eo_ref[...] = y

    return pl.pallas_call(
        body,
        grid=(NE // R,),
        in_specs=[
            pl.BlockSpec((4, R, 32), lambda i: (0, i, 0)),
            pl.BlockSpec((R, 8), lambda i: (i, 0)),
            pl.BlockSpec((R, 128), lambda i: (i, 0)),
        ],
        out_specs=[pl.BlockSpec((R, 128), lambda i: (i, 0))] * 2,
        out_shape=[jax.ShapeDtypeStruct((NE, 128), jnp.float32)] * 2,
    )(aggq, counts, res_in)


def _norm_user(aggq, res_in):
    R = 400

    def body(a_ref, r_ref, ro_ref):
        a = a_ref[...]
        x = jnp.concatenate([a[0], a[1], a[2], a[3]], axis=-1)
        n = jnp.sqrt(jnp.sum(x * x, axis=-1, keepdims=True))
        y = x / jnp.maximum(n, 1e-12)
        ro_ref[...] = r_ref[...] + y

    return pl.pallas_call(
        body,
        grid=(NU // R,),
        in_specs=[
            pl.BlockSpec((4, R, 32), lambda i: (0, i, 0)),
            pl.BlockSpec((R, 128), lambda i: (i, 0)),
        ],
        out_specs=pl.BlockSpec((R, 128), lambda i: (i, 0)),
        out_shape=jax.ShapeDtypeStruct((NU, 128), jnp.float32),
    )(aggq, res_in)


def _kg_pass(q, count_this, sid, eq, heads, tails, types, wflat, zeros32,
             zeros8, oe, ocnt, acc, accc, hbuf, tbuf, ybuf, rows, wv, onev,
             gsem):
    off = pl.multiple_of(jnp.minimum(sid * ES, ACC_E - ES), 8)
    pltpu.sync_copy(wflat.at[q], wv)
    pltpu.sync_copy(zeros32, acc.at[pl.ds(off, ES)])
    if count_this:
        pltpu.sync_copy(zeros8, accc.at[pl.ds(off, ES)])
    plsc.subcore_barrier()

    def chunk(j, carry):
        row = sid * ET + j
        pltpu.sync_copy(tails.at[row], tbuf)
        pltpu.sync_copy(heads.at[row], hbuf)
        pltpu.sync_copy(types.at[row], ybuf)
        pltpu.async_copy(eq.at[tbuf.at[0]], rows, gsem).wait()

        @plsc.parallel_loop(0, 128, step=16, unroll=2)
        def _mul(i):
            tv = ybuf[0, pl.ds(i, 16)] - 1
            for l in range(16):
                off = tv[l] * QW
                w0 = wv[pl.ds(off, 16)]
                w1 = wv[pl.ds(off + 16, 16)]
                rows[i + l, pl.ds(0, 16)] = rows[i + l, pl.ds(0, 16)] * w0
                rows[i + l, pl.ds(16, 16)] = rows[i + l, pl.ds(16, 16)] * w1

        pltpu.sync_copy(rows, acc.at[hbuf.at[0]], add=True)
        if count_this:
            pltpu.sync_copy(onev, accc.at[hbuf.at[0]], add=True)
        return carry

    lax.fori_loop(0, ET, chunk, 0)
    plsc.subcore_barrier()
    pltpu.sync_copy(acc.at[pl.ds(off, ES)], oe.at[pl.ds(off, ES)])
    if count_this:
        pltpu.sync_copy(accc.at[pl.ds(off, ES)], ocnt.at[pl.ds(off, ES)])
    plsc.subcore_barrier()


def _user_pass(q, sid, eq, irows, icols, ivals, zeros32, ou, acc, hbuf, tbuf,
               vbuf, rows, gsem):
    off = pl.multiple_of(jnp.minimum(sid * US, ACC_U - US), 8)
    pltpu.sync_copy(zeros32.at[pl.ds(0, US)], acc.at[pl.ds(off, US)])
    plsc.subcore_barrier()

    def chunk(j, carry):
        row = sid * IT + j
        pltpu.sync_copy(icols.at[row], tbuf)
        pltpu.sync_copy(irows.at[row], hbuf)
        pltpu.sync_copy(ivals.at[row], vbuf)
        pltpu.async_copy(eq.at[tbuf.at[0]], rows, gsem).wait()

        @plsc.parallel_loop(0, 128, step=16, unroll=2)
        def _mul(i):
            vv = vbuf[0, pl.ds(i, 16)]
            for l in range(16):
                v = vv[l]
                rows[i + l, pl.ds(0, 16)] = rows[i + l, pl.ds(0, 16)] * v
                rows[i + l, pl.ds(16, 16)] = rows[i + l, pl.ds(16, 16)] * v

        pltpu.sync_copy(rows, acc.at[hbuf.at[0]], add=True)
        return carry

    lax.fori_loop(0, IT, chunk, 0)
    plsc.subcore_barrier()
    pltpu.sync_copy(acc.at[pl.ds(off, US)], ou.at[pl.ds(off, US)])
    plsc.subcore_barrier()


def _sc_body(do_counts, eq0, eq1, eq2, eq3, heads, tails, types, irows,
             icols, ivals, wflat, ones8, zeros32, zeros8,
             oe0, oe1, oe2, oe3, ou0, ou1, ou2, ou3, ocnt,
             acc, accc, hbuf, tbuf, ybuf, vbuf, rows, wv, onev, gsem):
    cid = lax.axis_index("c")
    sid = lax.axis_index("s")
    eqs = (eq0, eq1, eq2, eq3)
    oes = (oe0, oe1, oe2, oe3)
    ous = (ou0, ou1, ou2, ou3)
    pltpu.sync_copy(ones8, onev)

    for q in range(4):
        count_this = do_counts and q == 0

        def _make(qq, cc):
            def _thunk():
                _kg_pass(qq, cc, sid, eqs[qq], heads, tails, types, wflat,
                         zeros32, zeros8, oes[qq], ocnt, acc, accc, hbuf,
                         tbuf, ybuf, rows, wv, onev, gsem)
                _user_pass(qq, sid, eqs[qq], irows, icols, ivals, zeros32,
                           ous[qq], acc, hbuf, tbuf, vbuf, rows, gsem)
            return _thunk

        pl.when(cid == q // 2)(_make(q, count_this))


@functools.lru_cache(maxsize=2)
def _sc_agg(do_counts):
    mesh = plsc.VectorSubcoreMesh(core_axis_name="c", subcore_axis_name="s",
                                  num_cores=2, num_subcores=16)
    out_type = (
        [jax.ShapeDtypeStruct((ACC_E, 32), jnp.float32)] * 4
        + [jax.ShapeDtypeStruct((ACC_U, 32), jnp.float32)] * 4
        + [jax.ShapeDtypeStruct((ACC_E, 8), jnp.float32)]
    )
    scratch = [
        pltpu.VMEM_SHARED((ACC_E, 32), jnp.float32),
        pltpu.VMEM_SHARED((ACC_E, 8), jnp.float32),
        pltpu.VMEM((1, 128), jnp.int32),
        pltpu.VMEM((1, 128), jnp.int32),
        pltpu.VMEM((1, 128), jnp.int32),
        pltpu.VMEM((1, 128), jnp.float32),
        pltpu.VMEM((128, 32), jnp.float32),
        pltpu.VMEM((352,), jnp.float32),
        pltpu.VMEM((128, 8), jnp.float32),
        pltpu.SemaphoreType.DMA,
    ]
    return pl.kernel(
        functools.partial(_sc_body, do_counts),
        out_type=out_type,
        mesh=mesh,
        scratch_types=scratch,
        compiler_params=pltpu.CompilerParams(use_tc_tiling_on_sc=False),
    )


def kernel(user_emb, entity_emb, edge_index, edge_type, interact_rows,
           interact_cols, interact_values, region_weight_mat, weight):
    head = edge_index[0]
    tail = edge_index[1]
    pe = EP - NEDGE
    heads_p = jnp.concatenate(
        [head, jnp.full((pe,), DUM_E, jnp.int32)]).reshape(ER, 1, 128)
    tails_p = jnp.concatenate(
        [tail, jnp.zeros((pe,), jnp.int32)]).reshape(ER, 1, 128)
    types_p = jnp.concatenate(
        [edge_type, jnp.ones((pe,), jnp.int32)]).reshape(ER, 1, 128)
    pi = IP - NNZ
    irows_p = jnp.concatenate(
        [interact_rows, jnp.full((pi,), DUM_U, jnp.int32)]).reshape(IR, 1, 128)
    icols_p = jnp.concatenate(
        [interact_cols, jnp.zeros((pi,), jnp.int32)]).reshape(IR, 1, 128)
    ivals_p = jnp.concatenate(
        [interact_values, jnp.zeros((pi,), jnp.float32)]).reshape(IR, 1, 128)
    wflat = weight.reshape(11, 4, 32).transpose(1, 0, 2).reshape(4, 352)
    ones8 = jnp.ones((128, 8), jnp.float32)
    zeros32 = jnp.zeros((ES, 32), jnp.float32)
    zeros8 = jnp.zeros((ES, 8), jnp.float32)
    rwm_pad = jnp.pad(region_weight_mat, ((0, RPAD - RD), (0, RPAD - RD)))

    ent = entity_emb
    e_res = entity_emb
    u_res = user_emb
    counts = None
    for hop in range(2):
        ent_pad = jnp.pad(ent[LO:HI], ((0, RPAD - RD), (0, 0)))
        e_region = _region_blend(rwm_pad, ent_pad)[:RD]
        e = jnp.concatenate([ent[:LO], e_region, ent[HI:]], axis=0)
        eq = e.reshape(NE, 4, 32).transpose(1, 0, 2)
        outs = _sc_agg(hop == 0)(
            eq[0], eq[1], eq[2], eq[3], heads_p, tails_p, types_p,
            irows_p, icols_p, ivals_p, wflat, ones8, zeros32, zeros8)
        oe0, oe1, oe2, oe3, ou0, ou1, ou2, ou3, ocnt = outs
        if hop == 0:
            counts = ocnt[:NE]
        eagg = jnp.stack([oe0[:NE], oe1[:NE], oe2[:NE], oe3[:NE]])
        uagg = jnp.stack([ou0[:NU], ou1[:NU], ou2[:NU], ou3[:NU]])
        e_res, ent = _norm_ent(eagg, counts, e_res)
        u_res = _norm_user(uagg, u_res)
    return e_res, u_res


# speedup vs baseline: 2.6476x; 1.7937x over previous
import functools

import jax
import jax.numpy as jnp
from jax import lax
from jax.experimental import pallas as pl
from jax.experimental.pallas import tpu as pltpu
from jax.experimental.pallas import tpu_sc as plsc

NE = 50000
NU = 20000
C = 128
QW = 32
LO, HI = 42033, 44630
RD = HI - LO
RPAD = 2688
NEDGE = 600000
NNZ = 500000
EP = 614400
IP = 512000
SUP = 10
ER = EP // 128 // SUP
IR = IP // 128 // SUP
ET = 30
IT = 25
CT = 15
ES = 3128
US = 1256
ACC_E = 50008
ACC_U = 20008
DUM_E = NE
DUM_U = NU


def _region_blend(rwm_pad, ent_pad):
    def body(a_ref, b_ref, s_ref, o_ref):
        mm = jnp.dot(a_ref[...], b_ref[...],
                     preferred_element_type=jnp.float32)
        o_ref[...] = 0.8 * s_ref[...] + 0.2 * mm

    return pl.pallas_call(
        body,
        grid=(RPAD // 128,),
        in_specs=[
            pl.BlockSpec((128, RPAD), lambda i: (i, 0)),
            pl.BlockSpec((RPAD, 128), lambda i: (0, 0)),
            pl.BlockSpec((128, 128), lambda i: (i, 0)),
        ],
        out_specs=pl.BlockSpec((128, 128), lambda i: (i, 0)),
        out_shape=jax.ShapeDtypeStruct((RPAD, 128), jnp.float32),
    )(rwm_pad, ent_pad, ent_pad)


def _norm_ent(aggq, cnt0, cnt1, res_in):
    R = 400

    def body(a_ref, c0_ref, c1_ref, r_ref, ro_ref, eo_ref):
        a = a_ref[...]
        x = jnp.concatenate([a[0], a[1], a[2], a[3]], axis=-1)
        cnt = jnp.maximum(c0_ref[...][:, 0:1] + c1_ref[...][:, 0:1], 1.0)
        x = x / cnt
        n = jnp.sqrt(jnp.sum(x * x, axis=-1, keepdims=True))
        y = x / jnp.maximum(n, 1e-12)
        ro_ref[...] = r_ref[...] + y
        eo_ref[...] = y

    return pl.pallas_call(
        body,
        grid=(NE // R,),
        in_specs=[
            pl.BlockSpec((4, R, 32), lambda i: (0, i, 0)),
            pl.BlockSpec((R, 32), lambda i: (i, 0)),
            pl.BlockSpec((R, 32), lambda i: (i, 0)),
            pl.BlockSpec((R, 128), lambda i: (i, 0)),
        ],
        out_specs=[pl.BlockSpec((R, 128), lambda i: (i, 0))] * 2,
        out_shape=[jax.ShapeDtypeStruct((NE, 128), jnp.float32)] * 2,
    )(aggq, cnt0, cnt1, res_in)


def _norm_user(aggq, res_in):
    R = 400

    def body(a_ref, r_ref, ro_ref):
        a = a_ref[...]
        x = jnp.concatenate([a[0], a[1], a[2], a[3]], axis=-1)
        n = jnp.sqrt(jnp.sum(x * x, axis=-1, keepdims=True))
        y = x / jnp.maximum(n, 1e-12)
        ro_ref[...] = r_ref[...] + y

    return pl.pallas_call(
        body,
        grid=(NU // R,),
        in_specs=[
            pl.BlockSpec((4, R, 32), lambda i: (0, i, 0)),
            pl.BlockSpec((R, 128), lambda i: (i, 0)),
        ],
        out_specs=pl.BlockSpec((R, 128), lambda i: (i, 0)),
        out_shape=jax.ShapeDtypeStruct((NU, 128), jnp.float32),
    )(aggq, res_in)


def _kg_pass(q, sid, eq, heads, tails, types, wflat, zeros32, oe, acc,
             tbuf, hbuf, ybuf, bufA, bufB, wv, sa, sb):
    off = pl.multiple_of(jnp.minimum(sid * ES, ACC_E - ES), 8)
    pltpu.sync_copy(wflat.at[q], wv)
    pltpu.sync_copy(zeros32, acc.at[pl.ds(off, ES)])
    plsc.subcore_barrier()

    def mul(buf, kk):
        @plsc.parallel_loop(0, 128, step=16, unroll=2)
        def _mul(i):
            tv = ybuf[kk, pl.ds(i, 16)] - 1
            for l in range(16):
                woff = tv[l] * QW
                w0 = wv[pl.ds(woff, 16)]
                w1 = wv[pl.ds(woff + 16, 16)]
                buf[i + l, pl.ds(0, 16)] = buf[i + l, pl.ds(0, 16)] * w0
                buf[i + l, pl.ds(16, 16)] = buf[i + l, pl.ds(16, 16)] * w1

    def super_body(s, carry):
        srow = sid * ET + s
        pltpu.sync_copy(tails.at[srow], tbuf)
        pltpu.sync_copy(heads.at[srow], hbuf)
        pltpu.sync_copy(types.at[srow], ybuf)
        pltpu.async_copy(eq.at[tbuf.at[0]], bufA, sa)

        def pair(j, c2):
            ka = 2 * j
            kb = 2 * j + 1
            pltpu.async_copy(eq.at[tbuf.at[kb]], bufB, sb)
            pltpu.make_async_copy(eq.at[tbuf.at[ka]], bufA, sa).wait()
            mul(bufA, ka)
            pltpu.sync_copy(bufA, acc.at[hbuf.at[ka]], add=True)

            @pl.when(j < SUP // 2 - 1)
            def _():
                pltpu.async_copy(eq.at[tbuf.at[ka + 2]], bufA, sa)

            pltpu.make_async_copy(eq.at[tbuf.at[kb]], bufB, sb).wait()
            mul(bufB, kb)
            pltpu.sync_copy(bufB, acc.at[hbuf.at[kb]], add=True)
            return c2

        lax.fori_loop(0, SUP // 2, pair, 0)
        return carry

    lax.fori_loop(0, ET, super_body, 0)
    plsc.subcore_barrier()
    pltpu.sync_copy(acc.at[pl.ds(off, ES)], oe.at[pl.ds(off, ES)])
    plsc.subcore_barrier()


def _user_pass(q, sid, eq, irows, icols, ivals, zeros32, ou, acc,
               tbuf, hbuf, vbuf, bufA, bufB, sa, sb):
    off = pl.multiple_of(jnp.minimum(sid * US, ACC_U - US), 8)
    pltpu.sync_copy(zeros32.at[pl.ds(0, US)], acc.at[pl.ds(off, US)])
    plsc.subcore_barrier()

    def mul(buf, kk):
        @plsc.parallel_loop(0, 128, step=16, unroll=2)
        def _mul(i):
            vv = vbuf[kk, pl.ds(i, 16)]
            for l in range(16):
                v = vv[l]
                buf[i + l, pl.ds(0, 16)] = buf[i + l, pl.ds(0, 16)] * v
                buf[i + l, pl.ds(16, 16)] = buf[i + l, pl.ds(16, 16)] * v

    def super_body(s, carry):
        srow = sid * IT + s
        pltpu.sync_copy(icols.at[srow], tbuf)
        pltpu.sync_copy(irows.at[srow], hbuf)
        pltpu.sync_copy(ivals.at[srow], vbuf)
        pltpu.async_copy(eq.at[tbuf.at[0]], bufA, sa)

        def pair(j, c2):
            ka = 2 * j
            kb = 2 * j + 1
            pltpu.async_copy(eq.at[tbuf.at[kb]], bufB, sb)
            pltpu.make_async_copy(eq.at[tbuf.at[ka]], bufA, sa).wait()
            mul(bufA, ka)
            pltpu.sync_copy(bufA, acc.at[hbuf.at[ka]], add=True)

            @pl.when(j < SUP // 2 - 1)
            def _():
                pltpu.async_copy(eq.at[tbuf.at[ka + 2]], bufA, sa)

            pltpu.make_async_copy(eq.at[tbuf.at[kb]], bufB, sb).wait()
            mul(bufB, kb)
            pltpu.sync_copy(bufB, acc.at[hbuf.at[kb]], add=True)
            return c2

        lax.fori_loop(0, SUP // 2, pair, 0)
        return carry

    lax.fori_loop(0, IT, super_body, 0)
    plsc.subcore_barrier()
    pltpu.sync_copy(acc.at[pl.ds(off, US)], ou.at[pl.ds(off, US)])
    plsc.subcore_barrier()


def _counts_pass(half, sid, heads, ones32, zeros32, oc, acc, hbuf, bufA):
    off = pl.multiple_of(jnp.minimum(sid * ES, ACC_E - ES), 8)
    pltpu.sync_copy(zeros32, acc.at[pl.ds(off, ES)])
    plsc.subcore_barrier()
    pltpu.sync_copy(ones32, bufA)

    def super_body(s, carry):
        srow = half * (ER // 2) + sid * CT + s
        pltpu.sync_copy(heads.at[srow], hbuf)
        for k in range(SUP):
            pltpu.sync_copy(bufA, acc.at[hbuf.at[k]], add=True)
        return carry

    lax.fori_loop(0, CT, super_body, 0)
    plsc.subcore_barrier()
    pltpu.sync_copy(acc.at[pl.ds(off, ES)], oc.at[pl.ds(off, ES)])
    plsc.subcore_barrier()


def _sc_body(do_counts, eq0, eq1, eq2, eq3, heads, tails, types, irows,
             icols, ivals, wflat, ones32, zeros32,
             oe0, oe1, oe2, oe3, ou0, ou1, ou2, ou3, oc0, oc1,
             acc, tbuf, hbuf, ybuf, vbuf, bufA, bufB, wv, sa, sb):
    cid = lax.axis_index("c")
    sid = lax.axis_index("s")
    eqs = (eq0, eq1, eq2, eq3)
    oes = (oe0, oe1, oe2, oe3)
    ous = (ou0, ou1, ou2, ou3)
    ocs = (oc0, oc1)

    for q in range(4):
        def _make(qq):
            def _thunk():
                if do_counts and qq % 2 == 0:
                    _counts_pass(qq // 2, sid, heads, ones32, zeros32,
                                 ocs[qq // 2], acc, hbuf, bufA)
                _kg_pass(qq, sid, eqs[qq], heads, tails, types, wflat,
                         zeros32, oes[qq], acc, tbuf, hbuf, ybuf, bufA,
                         bufB, wv, sa, sb)
                _user_pass(qq, sid, eqs[qq], irows, icols, ivals, zeros32,
                           ous[qq], acc, tbuf, hbuf, vbuf, bufA, bufB,
                           sa, sb)
            return _thunk

        pl.when(cid == q // 2)(_make(q))


@functools.lru_cache(maxsize=2)
def _sc_agg(do_counts):
    mesh = plsc.VectorSubcoreMesh(core_axis_name="c", subcore_axis_name="s",
                                  num_cores=2, num_subcores=16)
    out_type = (
        [jax.ShapeDtypeStruct((ACC_E, 32), jnp.float32)] * 4
        + [jax.ShapeDtypeStruct((ACC_U, 32), jnp.float32)] * 4
        + [jax.ShapeDtypeStruct((ACC_E, 32), jnp.float32)] * 2
    )
    scratch = [
        pltpu.VMEM_SHARED((ACC_E, 32), jnp.float32),
        pltpu.VMEM((SUP, 128), jnp.int32),
        pltpu.VMEM((SUP, 128), jnp.int32),
        pltpu.VMEM((SUP, 128), jnp.int32),
        pltpu.VMEM((SUP, 128), jnp.float32),
        pltpu.VMEM((128, 32), jnp.float32),
        pltpu.VMEM((128, 32), jnp.float32),
        pltpu.VMEM((352,), jnp.float32),
        pltpu.SemaphoreType.DMA,
        pltpu.SemaphoreType.DMA,
    ]
    return pl.kernel(
        functools.partial(_sc_body, do_counts),
        out_type=out_type,
        mesh=mesh,
        scratch_types=scratch,
        compiler_params=pltpu.CompilerParams(use_tc_tiling_on_sc=False),
    )


def kernel(user_emb, entity_emb, edge_index, edge_type, interact_rows,
           interact_cols, interact_values, region_weight_mat, weight):
    head = edge_index[0]
    tail = edge_index[1]
    pe = EP - NEDGE
    heads_p = jnp.concatenate(
        [head, jnp.full((pe,), DUM_E, jnp.int32)]).reshape(ER, SUP, 128)
    tails_p = jnp.concatenate(
        [tail, jnp.zeros((pe,), jnp.int32)]).reshape(ER, SUP, 128)
    types_p = jnp.concatenate(
        [edge_type, jnp.ones((pe,), jnp.int32)]).reshape(ER, SUP, 128)
    pi = IP - NNZ
    irows_p = jnp.concatenate(
        [interact_rows,
         jnp.full((pi,), DUM_U, jnp.int32)]).reshape(IR, SUP, 128)
    icols_p = jnp.concatenate(
        [interact_cols, jnp.zeros((pi,), jnp.int32)]).reshape(IR, SUP, 128)
    ivals_p = jnp.concatenate(
        [interact_values,
         jnp.zeros((pi,), jnp.float32)]).reshape(IR, SUP, 128)
    wflat = weight.reshape(11, 4, 32).transpose(1, 0, 2).reshape(4, 352)
    ones32 = jnp.ones((128, 32), jnp.float32)
    zeros32 = jnp.zeros((ES, 32), jnp.float32)
    rwm_pad = jnp.pad(region_weight_mat, ((0, RPAD - RD), (0, RPAD - RD)))

    ent = entity_emb
    e_res = entity_emb
    u_res = user_emb
    cnt0 = cnt1 = None
    for hop in range(2):
        ent_pad = jnp.pad(ent[LO:HI], ((0, RPAD - RD), (0, 0)))
        e_region = _region_blend(rwm_pad, ent_pad)[:RD]
        e = jnp.concatenate([ent[:LO], e_region, ent[HI:]], axis=0)
        eq = e.reshape(NE, 4, 32).transpose(1, 0, 2)
        outs = _sc_agg(hop == 0)(
            eq[0], eq[1], eq[2], eq[3], heads_p, tails_p, types_p,
            irows_p, icols_p, ivals_p, wflat, ones32, zeros32)
        oe0, oe1, oe2, oe3, ou0, ou1, ou2, ou3, oc0, oc1 = outs
        if hop == 0:
            cnt0 = oc0[:NE]
            cnt1 = oc1[:NE]
        eagg = jnp.stack([oe0[:NE], oe1[:NE], oe2[:NE], oe3[:NE]])
        uagg = jnp.stack([ou0[:NU], ou1[:NU], ou2[:NU], ou3[:NU]])
        e_res, ent = _norm_ent(eagg, cnt0, cnt1, e_res)
        u_res = _norm_user(uagg, u_res)
    return e_res, u_res
